# baseline (speedup 1.0000x reference)
import functools

import jax
import jax.numpy as jnp
from jax import lax
from jax.experimental import pallas as pl
from jax.experimental.pallas import tpu as pltpu
from jax.experimental.pallas import tpu_sc as plsc

_VOCAB = 1000000
_EMB_DIM = 64
_BATCH = 4096
_HIST = 50
_B = _BATCH * _HIST

_info = plsc.get_sparse_core_info()
_NC = _info.num_cores
_NS = _info.num_subcores
_NW = _NC * _NS
_B_PER_W = _B // _NW
_CHUNK = 800
_N_CHUNKS = _B_PER_W // _CHUNK

_mesh = plsc.VectorSubcoreMesh(core_axis_name="c", subcore_axis_name="s")


@functools.partial(
    pl.kernel,
    mesh=_mesh,
    out_type=jax.ShapeDtypeStruct((_B, _EMB_DIM), jnp.float32),
    scratch_types=[
        pltpu.VMEM((_CHUNK,), jnp.int32),
        pltpu.VMEM((_CHUNK, _EMB_DIM), jnp.float32),
        pltpu.SemaphoreType.DMA,
    ],
    compiler_params=pltpu.CompilerParams(use_tc_tiling_on_sc=False),
)
def _gather_kernel(table_hbm, idx_hbm, out_hbm, idx_v, rows_v, sem):
    wid = lax.axis_index("s") * _NC + lax.axis_index("c")
    base = wid * _B_PER_W
    for c in range(_N_CHUNKS):
        off = base + c * _CHUNK
        pltpu.sync_copy(idx_hbm.at[pl.ds(off, _CHUNK)], idx_v)
        pltpu.async_copy(table_hbm.at[idx_v], rows_v, sem).wait()
        pltpu.sync_copy(rows_v, out_hbm.at[pl.ds(off, _CHUNK)])


def kernel(x, table):
    out = _gather_kernel(table, x.reshape(-1))
    return out.reshape(_BATCH, _HIST, _EMB_DIM)


# speedup vs baseline: 1.0083x; 1.0083x over previous
import functools

import jax
import jax.numpy as jnp
from jax import lax
from jax.experimental import pallas as pl
from jax.experimental.pallas import tpu as pltpu
from jax.experimental.pallas import tpu_sc as plsc

_VOCAB = 1000000
_EMB_DIM = 64
_BATCH = 4096
_HIST = 50
_B = _BATCH * _HIST

_info = plsc.get_sparse_core_info()
_NC = _info.num_cores
_NS = _info.num_subcores
_NW = _NC * _NS
_B_PER_W = _B // _NW
_CHUNK = 640
_NBUF = 3
_N_CHUNKS = _B_PER_W // _CHUNK

_mesh = plsc.VectorSubcoreMesh(core_axis_name="c", subcore_axis_name="s")


@functools.partial(
    pl.kernel,
    mesh=_mesh,
    out_type=jax.ShapeDtypeStruct((_B, _EMB_DIM), jnp.float32),
    scratch_types=[
        [pltpu.VMEM((_CHUNK,), jnp.int32) for _ in range(_NBUF)],
        [pltpu.VMEM((_CHUNK, _EMB_DIM), jnp.float32) for _ in range(_NBUF)],
        [pltpu.SemaphoreType.DMA for _ in range(_NBUF)],
        [pltpu.SemaphoreType.DMA for _ in range(_NBUF)],
    ],
    compiler_params=pltpu.CompilerParams(use_tc_tiling_on_sc=False),
)
def _gather_kernel(table_hbm, idx_hbm, out_hbm, ib, rb, gsem, wsem):
    wid = lax.axis_index("s") * _NC + lax.axis_index("c")
    base = wid * _B_PER_W

    def start_gather(c):
        b = c % _NBUF
        pltpu.sync_copy(idx_hbm.at[pl.ds(base + c * _CHUNK, _CHUNK)], ib[b])
        pltpu.async_copy(table_hbm.at[ib[b]], rb[b], gsem[b])

    for c in range(min(_NBUF, _N_CHUNKS)):
        start_gather(c)

    for c in range(_N_CHUNKS):
        b = c % _NBUF
        pltpu.make_async_copy(table_hbm.at[ib[b]], rb[b], gsem[b]).wait()
        w = pltpu.async_copy(rb[b], out_hbm.at[pl.ds(base + c * _CHUNK, _CHUNK)], wsem[b])
        if c + _NBUF < _N_CHUNKS:
            w.wait()
            start_gather(c + _NBUF)

    for c in range(max(0, _N_CHUNKS - _NBUF), _N_CHUNKS):
        b = c % _NBUF
        pltpu.make_async_copy(rb[b], out_hbm.at[pl.ds(base + c * _CHUNK, _CHUNK)], wsem[b]).wait()


def kernel(x, table):
    out = _gather_kernel(table, x.reshape(-1))
    return out.reshape(_BATCH, _HIST, _EMB_DIM)
